# Initial kernel scaffold; baseline (speedup 1.0000x reference)
import jax
import jax.numpy as jnp
from jax.experimental import pallas as pl


def kernel(query_seq_length, key_seq_length, weight):
    raise NotImplementedError("write your pallas kernel here")


# speedup vs baseline: 41.5364x; 41.5364x over previous
import functools

import jax
import jax.numpy as jnp
from jax import lax
from jax.experimental import pallas as pl
from jax.experimental.pallas import tpu as pltpu
from jax.experimental.pallas import tpu_sc as plsc

H = 16
Q = 2048
K = 2048
RS = 4096
NCHUNK = RS // 16
GRP = 8


def _bucket16(d):
    rp = d - jnp.full((16,), 2047, jnp.int32)
    pos = jnp.where(rp > 0, jnp.full((16,), 16, jnp.int32),
                    jnp.zeros((16,), jnp.int32))
    a = jnp.abs(rp)
    ac = jnp.maximum(a, jnp.full((16,), 1, jnp.int32))
    bits = lax.bitcast_convert_type(ac.astype(jnp.float32), jnp.int32)
    e = (bits >> 23) - jnp.full((16,), 127, jnp.int32)
    t = (ac * ac >= lax.shift_left(jnp.full((16,), 1, jnp.int32),
                                   2 * e + 1)).astype(jnp.int32)
    bl = jnp.minimum(2 * e + t + jnp.full((16,), 2, jnp.int32),
                     jnp.full((16,), 15, jnp.int32))
    small = a < jnp.full((16,), 8, jnp.int32)
    return jnp.where(small, a, bl) + pos


def _rpe_body(w_hbm, out_hbm, w_v, table_v, sem):
    cid = lax.axis_index("c")
    sid = lax.axis_index("s")
    shift = 15 - sid

    pltpu.sync_copy(w_hbm, w_v)

    def build_chunk(cidx, carry):
        base = pl.multiple_of(cidx * 16, 16)
        d = lax.iota(jnp.int32, 16) + (base + shift)
        bkt = _bucket16(d)
        for h in range(H):
            vals = plsc.load_gather(w_v, [bkt, jnp.full((16,), h, jnp.int32)])
            table_v[pl.ds(h * RS + base, 16)] = vals
        return carry

    lax.fori_loop(0, NCHUNK, build_chunk, 0)

    qbase = sid + 16 * cid * 64

    def issue(n):
        h = n // 64
        i = n % 64
        q = qbase + 16 * i
        src = pl.multiple_of(h * RS + (2047 - q - shift), 16)
        dst = pl.multiple_of((h * Q + q) * K, K)
        return pltpu.make_async_copy(
            table_v.at[pl.ds(src, K)], out_hbm.at[pl.ds(dst, K)], sem)

    def group(g, carry):
        n0 = g * GRP
        cps = [issue(n0 + b) for b in range(GRP)]
        for cp in cps:
            cp.start()
        for cp in cps:
            cp.wait()
        return carry

    lax.fori_loop(0, (H * 64) // GRP, group, 0)


@jax.jit
def _rpe(weight):
    mesh = plsc.VectorSubcoreMesh(core_axis_name="c", subcore_axis_name="s")
    flat = pl.kernel(
        _rpe_body,
        out_type=jax.ShapeDtypeStruct((H * Q * K,), jnp.float32),
        mesh=mesh,
        compiler_params=pltpu.CompilerParams(needs_layout_passes=False),
        scratch_types=[
            pltpu.VMEM((32, H), jnp.float32),
            pltpu.VMEM((H * RS,), jnp.float32),
            pltpu.SemaphoreType.DMA,
        ],
    )(weight)
    return flat.reshape(1, H, Q, K)


def kernel(query_seq_length, key_seq_length, weight):
    del query_seq_length, key_seq_length
    return _rpe(weight)


# speedup vs baseline: 41.7186x; 1.0044x over previous
import functools

import jax
import jax.numpy as jnp
from jax import lax
from jax.experimental import pallas as pl
from jax.experimental.pallas import tpu as pltpu
from jax.experimental.pallas import tpu_sc as plsc

H = 16
Q = 2048
K = 2048
RS = 4096
NCHUNK = RS // 16
GRP = 8


def _bucket16(d):
    rp = d - jnp.full((16,), 2047, jnp.int32)
    pos = jnp.where(rp > 0, jnp.full((16,), 16, jnp.int32),
                    jnp.zeros((16,), jnp.int32))
    a = jnp.abs(rp)
    ac = jnp.maximum(a, jnp.full((16,), 1, jnp.int32))
    bits = lax.bitcast_convert_type(ac.astype(jnp.float32), jnp.int32)
    e = (bits >> 23) - jnp.full((16,), 127, jnp.int32)
    t = (ac * ac >= lax.shift_left(jnp.full((16,), 1, jnp.int32),
                                   2 * e + 1)).astype(jnp.int32)
    bl = jnp.minimum(2 * e + t + jnp.full((16,), 2, jnp.int32),
                     jnp.full((16,), 15, jnp.int32))
    small = a < jnp.full((16,), 8, jnp.int32)
    return jnp.where(small, a, bl) + pos


def _rpe_body(w_hbm, out_hbm, w_v, table_v, sem):
    cid = lax.axis_index("c")
    sid = lax.axis_index("s")
    shift = 15 - sid

    pltpu.sync_copy(w_hbm, w_v)

    def build_chunk(cidx, carry):
        base = pl.multiple_of(cidx * 16, 16)
        d = lax.iota(jnp.int32, 16) + (base + shift)
        bkt = _bucket16(d)
        for h in range(H):
            vals = plsc.load_gather(w_v, [bkt, jnp.full((16,), h, jnp.int32)])
            table_v[h, pl.ds(base, 16)] = vals
        return carry

    lax.fori_loop(0, NCHUNK, build_chunk, 0)

    qbase = sid + 16 * cid * 64

    def issue(i):
        q = qbase + 16 * i
        src = pl.multiple_of(2047 - q - shift, 16)
        dst = pl.multiple_of(q * K, K)
        return pltpu.make_async_copy(
            table_v.at[:, pl.ds(src, K)], out_hbm.at[:, pl.ds(dst, K)], sem)

    def group(g, carry):
        n0 = g * GRP
        cps = [issue(n0 + b) for b in range(GRP)]
        for cp in cps:
            cp.start()
        for cp in cps:
            cp.wait()
        return carry

    lax.fori_loop(0, 64 // GRP, group, 0)


@jax.jit
def _rpe(weight):
    mesh = plsc.VectorSubcoreMesh(core_axis_name="c", subcore_axis_name="s")
    flat = pl.kernel(
        _rpe_body,
        out_type=jax.ShapeDtypeStruct((H, Q * K), jnp.float32),
        mesh=mesh,
        compiler_params=pltpu.CompilerParams(
            needs_layout_passes=False, use_tc_tiling_on_sc=False),
        scratch_types=[
            pltpu.VMEM((32, H), jnp.float32),
            pltpu.VMEM((H, RS), jnp.float32),
            pltpu.SemaphoreType.DMA,
        ],
    )(weight)
    return flat.reshape(1, H, Q, K)


def kernel(query_seq_length, key_seq_length, weight):
    del query_seq_length, key_seq_length
    return _rpe(weight)
